# baseline (speedup 1.0000x reference)
import numpy as np
import jax
import jax.numpy as jnp
from jax.experimental import pallas as pl
from jax.experimental.pallas import tpu as pltpu

N = 10000
NFEAT = 128
NCLASSES = 16

MA = 400
NJ = N // MA
KB = 1024
TAILW = 384
TAILOFF = 9216
TAILROW = (NJ - 1) * MA


def _support1_kernel(x_ref, w1_ref, out_ref):
    out_ref[...] = jnp.dot(
        x_ref[...].astype(jnp.bfloat16), w1_ref[...].astype(jnp.bfloat16),
        preferred_element_type=jnp.float32).astype(jnp.bfloat16)


def _phasea_kernel(adj_ref, s1_ref, b1_ref, w2_ref, b2_ref,
                   s2_ref, opart_ref, s2v_ref):
    i = pl.program_id(0)
    j = NJ - 1 - i
    adj_bf = adj_ref[...].astype(jnp.bfloat16)

    @pl.when(i == 0)
    def _init():
        s2v_ref[...] = jnp.zeros_like(s2v_ref)

    acc = jnp.dot(adj_bf, s1_ref[...], preferred_element_type=jnp.float32)
    feat = jnp.maximum(acc + b1_ref[...], 0.0)
    s2j = jnp.dot(feat.astype(jnp.bfloat16), w2_ref[...],
                  preferred_element_type=jnp.float32).astype(jnp.bfloat16)
    s2v_ref[pl.ds(j * MA, MA), :] = s2j
    s2_ref[...] = s2j
    opart_ref[...] = jnp.dot(adj_bf, s2v_ref[...],
                             preferred_element_type=jnp.float32) + b2_ref[...]


def _schedule_b():
    steps = [(0, 0, 1, 1)]
    for j in range(1, NJ):
        kmax = -(-(MA * j) // KB) if j < NJ - 1 else TAILOFF // KB
        for k in range(kmax):
            steps.append((j, k, 1 if k == 0 else 0,
                          1 if (k == kmax - 1 and j < NJ - 1) else 0))
    steps.append((NJ - 1, steps[-1][1], 0, 1))
    im = np.array([s[0] for s in steps], np.int32)
    km = np.array([s[1] for s in steps], np.int32)
    fm = np.array([s[2] for s in steps], np.int32)
    lm = np.array([s[3] for s in steps], np.int32)
    return im, km, fm, lm


_IM, _KM, _FM, _LM = _schedule_b()
TOTAL_B = len(_IM)


def _phaseb_kernel(im_ref, km_ref, fm_ref, lm_ref,
                   adj_ref, adjt_ref, s2_ref, opart_ref, out_ref,
                   acc_ref, s2r_ref):
    t = pl.program_id(0)
    j = im_ref[t]
    k = km_ref[t]
    first = fm_ref[t]
    last = lm_ref[t]

    @pl.when(t == 0)
    def _init():
        s2r_ref[...] = jnp.zeros_like(s2r_ref)

    @pl.when(jnp.logical_and(first == 1, j > 0))
    def _fill():
        base = (j - 1) * MA
        s2r_ref[pl.ds(base, MA), :] = s2_ref[pl.ds(base, MA), :]

    is_tail = t == TOTAL_B - 1

    @pl.when(jnp.logical_not(is_tail))
    def _regular():
        c = jnp.dot(adj_ref[...].astype(jnp.bfloat16),
                    s2r_ref[pl.ds(k * KB, KB), :],
                    preferred_element_type=jnp.float32)

        @pl.when(first == 1)
        def _set():
            acc_ref[...] = opart_ref[...] + c

        @pl.when(first == 0)
        def _add():
            acc_ref[...] += c

    @pl.when(is_tail)
    def _tail():
        acc_ref[...] += jnp.dot(adjt_ref[...].astype(jnp.bfloat16),
                                s2r_ref[pl.ds(TAILOFF, TAILW), :],
                                preferred_element_type=jnp.float32)

    @pl.when(last == 1)
    def _write():
        out_ref[...] = acc_ref[...]


@jax.jit
def kernel(x, adj, W1, b1, W2, b2):
    b1r = b1.reshape(1, NFEAT)
    b2r = b2.reshape(1, NCLASSES)
    w2_bf = W2.astype(jnp.bfloat16)

    support1 = pl.pallas_call(
        _support1_kernel,
        grid=(5,),
        in_specs=[
            pl.BlockSpec((N // 5, NFEAT), lambda i: (i, 0)),
            pl.BlockSpec((NFEAT, NFEAT), lambda i: (0, 0)),
        ],
        out_specs=pl.BlockSpec((N // 5, NFEAT), lambda i: (i, 0)),
        out_shape=jax.ShapeDtypeStruct((N, NFEAT), jnp.bfloat16),
    )(x, W1)

    s2, opart = pl.pallas_call(
        _phasea_kernel,
        grid=(NJ,),
        in_specs=[
            pl.BlockSpec((MA, N), lambda i: (NJ - 1 - i, 0)),
            pl.BlockSpec((N, NFEAT), lambda i: (0, 0)),
            pl.BlockSpec((1, NFEAT), lambda i: (0, 0)),
            pl.BlockSpec((NFEAT, NCLASSES), lambda i: (0, 0)),
            pl.BlockSpec((1, NCLASSES), lambda i: (0, 0)),
        ],
        out_specs=[
            pl.BlockSpec((MA, NCLASSES), lambda i: (NJ - 1 - i, 0)),
            pl.BlockSpec((MA, NCLASSES), lambda i: (NJ - 1 - i, 0)),
        ],
        out_shape=[
            jax.ShapeDtypeStruct((N, NCLASSES), jnp.bfloat16),
            jax.ShapeDtypeStruct((N, NCLASSES), jnp.float32),
        ],
        scratch_shapes=[pltpu.VMEM((N, NCLASSES), jnp.bfloat16)],
        compiler_params=pltpu.CompilerParams(
            dimension_semantics=("arbitrary",)),
    )(adj, support1, b1r, w2_bf, b2r)

    grid_spec = pltpu.PrefetchScalarGridSpec(
        num_scalar_prefetch=4,
        grid=(TOTAL_B,),
        in_specs=[
            pl.BlockSpec((MA, KB), lambda t, im, km, fm, lm: (im[t], km[t])),
            pl.BlockSpec((MA, TAILW),
                         lambda t, im, km, fm, lm: (NJ - 1, TAILOFF // TAILW)),
            pl.BlockSpec((N, NCLASSES), lambda t, im, km, fm, lm: (0, 0)),
            pl.BlockSpec((MA, NCLASSES), lambda t, im, km, fm, lm: (im[t], 0)),
        ],
        out_specs=pl.BlockSpec((MA, NCLASSES),
                               lambda t, im, km, fm, lm: (im[t], 0)),
        scratch_shapes=[
            pltpu.VMEM((MA, NCLASSES), jnp.float32),
            pltpu.VMEM((N, NCLASSES), jnp.bfloat16),
        ],
    )

    logit = pl.pallas_call(
        _phaseb_kernel,
        grid_spec=grid_spec,
        out_shape=jax.ShapeDtypeStruct((N, NCLASSES), jnp.float32),
        compiler_params=pltpu.CompilerParams(
            dimension_semantics=("arbitrary",)),
    )(jnp.asarray(_IM), jnp.asarray(_KM), jnp.asarray(_FM), jnp.asarray(_LM),
      adj, adj, s2, opart)

    return logit


# speedup vs baseline: 1.4254x; 1.4254x over previous
import numpy as np
import jax
import jax.numpy as jnp
from jax.experimental import pallas as pl
from jax.experimental.pallas import tpu as pltpu

N = 10000
NFEAT = 128
NCLASSES = 16

MA = 400
NJ = N // MA
KB = 1024
NFULL = 2


def _support1_kernel(x_ref, w1_ref, out_ref):
    out_ref[...] = jnp.dot(x_ref[...], w1_ref[...],
                           preferred_element_type=jnp.float32)


def _phasea_kernel(adj_ref, s1_ref, b1_ref, w2_ref, b2_ref,
                   s2_ref, opart_ref, rhs_ref):
    i = pl.program_id(0)
    j = NJ - 1 - i

    @pl.when(i == 0)
    def _init():
        rhs_ref[:, :NFEAT] = s1_ref[...]
        rhs_ref[:, NFEAT:] = jnp.zeros((N, NFEAT), jnp.float32)

    both = jnp.dot(adj_ref[...], rhs_ref[...],
                   preferred_element_type=jnp.float32)
    feat = jnp.maximum(both[:, :NFEAT] + b1_ref[...], 0.0)
    s2j = jnp.dot(feat, w2_ref[...], preferred_element_type=jnp.float32)
    rhs_ref[pl.ds(j * MA, MA), NFEAT:] = s2j
    s2_ref[...] = s2j
    opart_ref[...] = both[:, NFEAT:NFEAT + NCLASSES] + b2_ref[...]


def _schedule_b():
    steps = []
    k_hold = 0
    for j in range(NJ - NFULL):
        kmax = -(-(MA * (j + 1)) // KB)
        for k in range(kmax):
            steps.append((j, k, 1 if k == 0 else 0,
                          1 if k == kmax - 1 else 0, 0))
            k_hold = k
    for j in range(NJ - NFULL, NJ):
        steps.append((j, k_hold, 1, 1, 1))
    jm = np.array([s[0] for s in steps], np.int32)
    km = np.array([s[1] for s in steps], np.int32)
    fm = np.array([s[2] for s in steps], np.int32)
    lm = np.array([s[3] for s in steps], np.int32)
    sm = np.array([s[4] for s in steps], np.int32)
    jf = np.array([NJ - NFULL] * (len(steps) - 1) + [NJ - 1], np.int32)
    return jm, km, fm, lm, sm, jf


_JM, _KM, _FM, _LM, _SM, _JF = _schedule_b()
TOTAL_B = len(_JM)


def _phaseb_kernel(jm_ref, km_ref, fm_ref, lm_ref, sm_ref, jf_ref,
                   adj_ref, adjf_ref, s2_ref, opart_ref, out_ref,
                   acc_ref, s2r_ref):
    t = pl.program_id(0)
    j = jm_ref[t]
    k = km_ref[t]
    first = fm_ref[t]
    last = lm_ref[t]
    full = sm_ref[t]

    @pl.when(t == 0)
    def _init():
        s2r_ref[...] = jnp.zeros_like(s2r_ref)

    @pl.when(first == 1)
    def _fill():
        base = j * MA
        s2r_ref[pl.ds(base, MA), :] = s2_ref[pl.ds(base, MA), :]

    @pl.when(full == 0)
    def _regular():
        c = jnp.dot(adj_ref[...], s2r_ref[pl.ds(k * KB, KB), :],
                    preferred_element_type=jnp.float32)[:, :NCLASSES]

        @pl.when(first == 1)
        def _set():
            acc_ref[...] = opart_ref[...] + c

        @pl.when(first == 0)
        def _add():
            acc_ref[...] += c

    @pl.when(full == 1)
    def _fullstep():
        c = jnp.dot(adjf_ref[...], s2r_ref[...],
                    preferred_element_type=jnp.float32)[:, :NCLASSES]
        acc_ref[...] = opart_ref[...] + c

    @pl.when(last == 1)
    def _write():
        out_ref[...] = acc_ref[...]


@jax.jit
def kernel(x, adj, W1, b1, W2, b2):
    b1r = b1.reshape(1, NFEAT)
    b2r = b2.reshape(1, NCLASSES)
    w2p = jnp.pad(W2, ((0, 0), (0, NFEAT - NCLASSES)))

    support1 = pl.pallas_call(
        _support1_kernel,
        grid=(5,),
        in_specs=[
            pl.BlockSpec((N // 5, NFEAT), lambda i: (i, 0)),
            pl.BlockSpec((NFEAT, NFEAT), lambda i: (0, 0)),
        ],
        out_specs=pl.BlockSpec((N // 5, NFEAT), lambda i: (i, 0)),
        out_shape=jax.ShapeDtypeStruct((N, NFEAT), jnp.float32),
    )(x, W1)

    s2, opart = pl.pallas_call(
        _phasea_kernel,
        grid=(NJ,),
        in_specs=[
            pl.BlockSpec((MA, N), lambda i: (NJ - 1 - i, 0)),
            pl.BlockSpec((N, NFEAT), lambda i: (0, 0)),
            pl.BlockSpec((1, NFEAT), lambda i: (0, 0)),
            pl.BlockSpec((NFEAT, NFEAT), lambda i: (0, 0)),
            pl.BlockSpec((1, NCLASSES), lambda i: (0, 0)),
        ],
        out_specs=[
            pl.BlockSpec((MA, NFEAT), lambda i: (NJ - 1 - i, 0)),
            pl.BlockSpec((MA, NCLASSES), lambda i: (NJ - 1 - i, 0)),
        ],
        out_shape=[
            jax.ShapeDtypeStruct((N, NFEAT), jnp.float32),
            jax.ShapeDtypeStruct((N, NCLASSES), jnp.float32),
        ],
        scratch_shapes=[pltpu.VMEM((N, 2 * NFEAT), jnp.float32)],
        compiler_params=pltpu.CompilerParams(
            dimension_semantics=("arbitrary",)),
    )(adj, support1, b1r, w2p, b2r)

    grid_spec = pltpu.PrefetchScalarGridSpec(
        num_scalar_prefetch=6,
        grid=(TOTAL_B,),
        in_specs=[
            pl.BlockSpec((MA, KB),
                         lambda t, jm, km, fm, lm, sm, jf: (jm[t], km[t])),
            pl.BlockSpec((MA, N),
                         lambda t, jm, km, fm, lm, sm, jf: (jf[t], 0)),
            pl.BlockSpec((N, NFEAT),
                         lambda t, jm, km, fm, lm, sm, jf: (0, 0)),
            pl.BlockSpec((MA, NCLASSES),
                         lambda t, jm, km, fm, lm, sm, jf: (jm[t], 0)),
        ],
        out_specs=pl.BlockSpec((MA, NCLASSES),
                               lambda t, jm, km, fm, lm, sm, jf: (jm[t], 0)),
        scratch_shapes=[
            pltpu.VMEM((MA, NCLASSES), jnp.float32),
            pltpu.VMEM((N, NFEAT), jnp.float32),
        ],
    )

    logit = pl.pallas_call(
        _phaseb_kernel,
        grid_spec=grid_spec,
        out_shape=jax.ShapeDtypeStruct((N, NCLASSES), jnp.float32),
        compiler_params=pltpu.CompilerParams(
            dimension_semantics=("arbitrary",)),
    )(jnp.asarray(_JM), jnp.asarray(_KM), jnp.asarray(_FM), jnp.asarray(_LM),
      jnp.asarray(_SM), jnp.asarray(_JF), adj, adj, s2, opart)

    return logit


# speedup vs baseline: 1.6131x; 1.1317x over previous
import numpy as np
import jax
import jax.numpy as jnp
from jax.experimental import pallas as pl
from jax.experimental.pallas import tpu as pltpu

N = 10000
NFEAT = 128
NCLASSES = 16

MA = 400
NJ = N // MA
KB = 2048
NFULL = 5


def _support1_kernel(x_ref, w1_ref, out_ref):
    out_ref[...] = jnp.dot(x_ref[...], w1_ref[...],
                           preferred_element_type=jnp.float32)


def _phasea_kernel(adj_ref, s1_ref, b1_ref, w2_ref, b2_ref,
                   s2_ref, opart_ref, rhs_ref):
    i = pl.program_id(0)
    j = NJ - 1 - i

    @pl.when(i == 0)
    def _init():
        rhs_ref[:, :NFEAT] = s1_ref[...]
        rhs_ref[:, NFEAT:] = jnp.zeros((N, NFEAT), jnp.float32)

    both = jnp.dot(adj_ref[...], rhs_ref[...],
                   preferred_element_type=jnp.float32)
    feat = jnp.maximum(both[:, :NFEAT] + b1_ref[...], 0.0)
    s2j = jnp.dot(feat, w2_ref[...], preferred_element_type=jnp.float32)
    rhs_ref[pl.ds(j * MA, MA), NFEAT:] = s2j
    s2_ref[...] = s2j
    opart_ref[...] = both[:, NFEAT:NFEAT + NCLASSES] + b2_ref[...]


def _schedule_b():
    steps = []
    k_hold = 0
    for j in range(NJ - NFULL):
        kmax = -(-(MA * (j + 1)) // KB)
        for k in range(kmax):
            steps.append((j, k, 1 if k == 0 else 0,
                          1 if k == kmax - 1 else 0, 0))
            k_hold = k
    for j in range(NJ - NFULL, NJ):
        steps.append((j, k_hold, 1, 1, 1))
    jm = np.array([s[0] for s in steps], np.int32)
    km = np.array([s[1] for s in steps], np.int32)
    fm = np.array([s[2] for s in steps], np.int32)
    lm = np.array([s[3] for s in steps], np.int32)
    sm = np.array([s[4] for s in steps], np.int32)
    jf = np.array([s[0] if s[4] == 1 else NJ - NFULL for s in steps],
                  np.int32)
    return jm, km, fm, lm, sm, jf


_JM, _KM, _FM, _LM, _SM, _JF = _schedule_b()
TOTAL_B = len(_JM)


def _phaseb_kernel(jm_ref, km_ref, fm_ref, lm_ref, sm_ref, jf_ref,
                   adj_ref, adjf_ref, s2_ref, opart_ref, out_ref,
                   acc_ref, s2r_ref):
    t = pl.program_id(0)
    j = jm_ref[t]
    k = km_ref[t]
    first = fm_ref[t]
    last = lm_ref[t]
    full = sm_ref[t]

    @pl.when(t == 0)
    def _init():
        s2r_ref[...] = jnp.zeros_like(s2r_ref)

    @pl.when(first == 1)
    def _fill():
        base = j * MA
        s2r_ref[pl.ds(base, MA), :] = s2_ref[pl.ds(base, MA), :]

    @pl.when(full == 0)
    def _regular():
        c = jnp.dot(adj_ref[...], s2r_ref[pl.ds(k * KB, KB), :],
                    preferred_element_type=jnp.float32)[:, :NCLASSES]

        @pl.when(first == 1)
        def _set():
            acc_ref[...] = opart_ref[...] + c

        @pl.when(first == 0)
        def _add():
            acc_ref[...] += c

    @pl.when(full == 1)
    def _fullstep():
        c = jnp.dot(adjf_ref[...], s2r_ref[...],
                    preferred_element_type=jnp.float32)[:, :NCLASSES]
        acc_ref[...] = opart_ref[...] + c

    @pl.when(last == 1)
    def _write():
        out_ref[...] = acc_ref[...]


@jax.jit
def kernel(x, adj, W1, b1, W2, b2):
    b1r = b1.reshape(1, NFEAT)
    b2r = b2.reshape(1, NCLASSES)
    w2p = jnp.pad(W2, ((0, 0), (0, NFEAT - NCLASSES)))

    support1 = pl.pallas_call(
        _support1_kernel,
        grid=(5,),
        in_specs=[
            pl.BlockSpec((N // 5, NFEAT), lambda i: (i, 0)),
            pl.BlockSpec((NFEAT, NFEAT), lambda i: (0, 0)),
        ],
        out_specs=pl.BlockSpec((N // 5, NFEAT), lambda i: (i, 0)),
        out_shape=jax.ShapeDtypeStruct((N, NFEAT), jnp.float32),
    )(x, W1)

    s2, opart = pl.pallas_call(
        _phasea_kernel,
        grid=(NJ,),
        in_specs=[
            pl.BlockSpec((MA, N), lambda i: (NJ - 1 - i, 0)),
            pl.BlockSpec((N, NFEAT), lambda i: (0, 0)),
            pl.BlockSpec((1, NFEAT), lambda i: (0, 0)),
            pl.BlockSpec((NFEAT, NFEAT), lambda i: (0, 0)),
            pl.BlockSpec((1, NCLASSES), lambda i: (0, 0)),
        ],
        out_specs=[
            pl.BlockSpec((MA, NFEAT), lambda i: (NJ - 1 - i, 0)),
            pl.BlockSpec((MA, NCLASSES), lambda i: (NJ - 1 - i, 0)),
        ],
        out_shape=[
            jax.ShapeDtypeStruct((N, NFEAT), jnp.float32),
            jax.ShapeDtypeStruct((N, NCLASSES), jnp.float32),
        ],
        scratch_shapes=[pltpu.VMEM((N, 2 * NFEAT), jnp.float32)],
        compiler_params=pltpu.CompilerParams(
            dimension_semantics=("arbitrary",)),
    )(adj, support1, b1r, w2p, b2r)

    grid_spec = pltpu.PrefetchScalarGridSpec(
        num_scalar_prefetch=6,
        grid=(TOTAL_B,),
        in_specs=[
            pl.BlockSpec((MA, KB),
                         lambda t, jm, km, fm, lm, sm, jf: (jm[t], km[t])),
            pl.BlockSpec((MA, N),
                         lambda t, jm, km, fm, lm, sm, jf: (jf[t], 0)),
            pl.BlockSpec((N, NFEAT),
                         lambda t, jm, km, fm, lm, sm, jf: (0, 0)),
            pl.BlockSpec((MA, NCLASSES),
                         lambda t, jm, km, fm, lm, sm, jf: (jm[t], 0)),
        ],
        out_specs=pl.BlockSpec((MA, NCLASSES),
                               lambda t, jm, km, fm, lm, sm, jf: (jm[t], 0)),
        scratch_shapes=[
            pltpu.VMEM((MA, NCLASSES), jnp.float32),
            pltpu.VMEM((N, NFEAT), jnp.float32),
        ],
    )

    logit = pl.pallas_call(
        _phaseb_kernel,
        grid_spec=grid_spec,
        out_shape=jax.ShapeDtypeStruct((N, NCLASSES), jnp.float32),
        compiler_params=pltpu.CompilerParams(
            dimension_semantics=("arbitrary",)),
    )(jnp.asarray(_JM), jnp.asarray(_KM), jnp.asarray(_FM), jnp.asarray(_LM),
      jnp.asarray(_SM), jnp.asarray(_JF), adj, adj, s2, opart)

    return logit


# speedup vs baseline: 1.6856x; 1.0449x over previous
import numpy as np
import jax
import jax.numpy as jnp
from jax.experimental import pallas as pl
from jax.experimental.pallas import tpu as pltpu

N = 10000
NFEAT = 128
NCLASSES = 16

MA = 400
NJ = N // MA
W4 = 4096
W8 = 8192
W1K = 1024
WT = 384
DIAG = (NJ - 1) * MA


def _phasea_kernel(adj_ref, x_ref, w1_ref, b1_ref, w2_ref, b2_ref,
                   s2_ref, opart_ref, rhs_ref):
    i = pl.program_id(0)
    j = NJ - 1 - i

    @pl.when(i == 0)
    def _init():
        rhs_ref[:, :NFEAT] = jnp.dot(x_ref[...], w1_ref[...],
                                     preferred_element_type=jnp.float32)
        rhs_ref[:, NFEAT:] = jnp.zeros((N, NFEAT), jnp.float32)

    both = jnp.dot(adj_ref[...], rhs_ref[...],
                   preferred_element_type=jnp.float32)
    feat = jnp.maximum(both[:, :NFEAT] + b1_ref[...], 0.0)
    s2j = jnp.dot(feat, w2_ref[...], preferred_element_type=jnp.float32)
    rhs_ref[pl.ds(j * MA, MA), NFEAT:] = s2j
    s2_ref[...] = s2j
    base = both[:, NFEAT:NFEAT + NCLASSES] + b2_ref[...]

    @pl.when(j != NJ - 1)
    def _plain():
        opart_ref[...] = base

    @pl.when(j == NJ - 1)
    def _with_diag():
        opart_ref[...] = base + jnp.dot(
            adj_ref[:, DIAG:], s2j,
            preferred_element_type=jnp.float32)[:, :NCLASSES]


def _schedule_b():
    steps = []
    for j in range(NJ):
        need = MA * (j + 1)
        row = [0 if need <= W4 else 1]
        if need > W8:
            row.append(2)
        if need > W8 + W1K:
            row.append(3)
        steps.extend((j, m) for m in row)
    jm = np.array([s[0] for s in steps], np.int32)
    md = np.array([s[1] for s in steps], np.int32)
    plans = []
    for mode, first_row in ((0, 0), (1, 10), (2, 20), (3, 23)):
        cur = first_row
        plan = []
        for j, m in steps:
            if m == mode:
                cur = j
            plan.append(cur)
        plans.append(np.array(plan, np.int32))
    return jm, md, plans[0], plans[1], plans[2], plans[3]


_JM, _MD, _J4, _J8, _J1, _JT = _schedule_b()
TOTAL_B = len(_JM)


def _phaseb_kernel(jm_ref, md_ref, j4_ref, j8_ref, j1_ref, jt_ref,
                   adj4_ref, adj8_ref, adj1_ref, adjt_ref, s2_ref, opart_ref,
                   out_ref, s2r_ref):
    t = pl.program_id(0)
    j = jm_ref[t]
    md = md_ref[t]

    @pl.when(t == 0)
    def _init():
        s2r_ref[...] = jnp.zeros_like(s2r_ref)

    @pl.when(md <= 1)
    def _fill():
        base = j * MA
        s2r_ref[pl.ds(base, MA), :] = s2_ref[pl.ds(base, MA), :]

    @pl.when(md == 0)
    def _m0():
        out_ref[...] = opart_ref[...] + jnp.dot(
            adj4_ref[...], s2r_ref[pl.ds(0, W4), :],
            preferred_element_type=jnp.float32)[:, :NCLASSES]

    @pl.when(md == 1)
    def _m1():
        out_ref[...] = opart_ref[...] + jnp.dot(
            adj8_ref[...], s2r_ref[pl.ds(0, W8), :],
            preferred_element_type=jnp.float32)[:, :NCLASSES]

    @pl.when(md == 2)
    def _m2():
        out_ref[...] += jnp.dot(
            adj1_ref[...], s2r_ref[pl.ds(W8, W1K), :],
            preferred_element_type=jnp.float32)[:, :NCLASSES]

    @pl.when(md == 3)
    def _m3():
        out_ref[...] += jnp.dot(
            adjt_ref[...], s2r_ref[pl.ds(W8 + W1K, WT), :],
            preferred_element_type=jnp.float32)[:, :NCLASSES]


@jax.jit
def kernel(x, adj, W1, b1, W2, b2):
    b1r = b1.reshape(1, NFEAT)
    b2r = b2.reshape(1, NCLASSES)
    w2p = jnp.pad(W2, ((0, 0), (0, NFEAT - NCLASSES)))

    s2, opart = pl.pallas_call(
        _phasea_kernel,
        grid=(NJ,),
        in_specs=[
            pl.BlockSpec((MA, N), lambda i: (NJ - 1 - i, 0)),
            pl.BlockSpec((N, NFEAT), lambda i: (0, 0)),
            pl.BlockSpec((NFEAT, NFEAT), lambda i: (0, 0)),
            pl.BlockSpec((1, NFEAT), lambda i: (0, 0)),
            pl.BlockSpec((NFEAT, NFEAT), lambda i: (0, 0)),
            pl.BlockSpec((1, NCLASSES), lambda i: (0, 0)),
        ],
        out_specs=[
            pl.BlockSpec((MA, NFEAT), lambda i: (NJ - 1 - i, 0)),
            pl.BlockSpec((MA, NCLASSES), lambda i: (NJ - 1 - i, 0)),
        ],
        out_shape=[
            jax.ShapeDtypeStruct((N, NFEAT), jnp.float32),
            jax.ShapeDtypeStruct((N, NCLASSES), jnp.float32),
        ],
        scratch_shapes=[pltpu.VMEM((N, 2 * NFEAT), jnp.float32)],
        compiler_params=pltpu.CompilerParams(
            dimension_semantics=("arbitrary",)),
    )(adj, x, W1, b1r, w2p, b2r)

    grid_spec = pltpu.PrefetchScalarGridSpec(
        num_scalar_prefetch=6,
        grid=(TOTAL_B,),
        in_specs=[
            pl.BlockSpec((MA, W4),
                         lambda t, jm, md, j4, j8, j1, jt: (j4[t], 0)),
            pl.BlockSpec((MA, W8),
                         lambda t, jm, md, j4, j8, j1, jt: (j8[t], 0)),
            pl.BlockSpec((MA, W1K),
                         lambda t, jm, md, j4, j8, j1, jt: (j1[t], W8 // W1K)),
            pl.BlockSpec((MA, WT),
                         lambda t, jm, md, j4, j8, j1, jt:
                         (jt[t], (W8 + W1K) // WT)),
            pl.BlockSpec((N, NFEAT),
                         lambda t, jm, md, j4, j8, j1, jt: (0, 0)),
            pl.BlockSpec((MA, NCLASSES),
                         lambda t, jm, md, j4, j8, j1, jt: (jm[t], 0)),
        ],
        out_specs=pl.BlockSpec((MA, NCLASSES),
                               lambda t, jm, md, j4, j8, j1, jt: (jm[t], 0)),
        scratch_shapes=[
            pltpu.VMEM((N, NFEAT), jnp.float32),
        ],
    )

    logit = pl.pallas_call(
        _phaseb_kernel,
        grid_spec=grid_spec,
        out_shape=jax.ShapeDtypeStruct((N, NCLASSES), jnp.float32),
        compiler_params=pltpu.CompilerParams(
            dimension_semantics=("arbitrary",)),
    )(jnp.asarray(_JM), jnp.asarray(_MD), jnp.asarray(_J4), jnp.asarray(_J8),
      jnp.asarray(_J1), jnp.asarray(_JT), adj, adj, adj, adj, s2, opart)

    return logit
